# baseline (speedup 1.0000x reference)
import functools
import math

import jax
import jax.numpy as jnp
from jax import lax
from jax.experimental import pallas as pl
from jax.experimental.pallas import tpu as pltpu
from jax.experimental.pallas import tpu_sc as plsc

_BATCH = 16384
_C = 10.0
_D = 400.0
_K = math.log(_C) / _D

_NUM_WORKERS = 32
_CHUNK = _BATCH // _NUM_WORKERS
_HALF = _CHUNK // 2
_LANES = 16


@functools.partial(
    pl.kernel,
    mesh=plsc.VectorSubcoreMesh(core_axis_name="c", subcore_axis_name="s"),
    out_type=jax.ShapeDtypeStruct((_BATCH,), jnp.float32),
    scratch_types=[
        pltpu.VMEM((_CHUNK,), jnp.int32),
        pltpu.VMEM((_CHUNK,), jnp.int32),
        pltpu.VMEM((_CHUNK,), jnp.float32),
        pltpu.VMEM((_CHUNK,), jnp.float32),
        pltpu.VMEM((_CHUNK,), jnp.float32),
        pltpu.SemaphoreType.DMA,
        pltpu.SemaphoreType.DMA,
    ],
)
def _elo_sc(matches_hbm, rating_hbm, out_hbm,
            idx_h, idx_a, gath_h, gath_a, out_v, s_h, s_a):
    wid = lax.axis_index("s") * 2 + lax.axis_index("c")
    base = wid * _CHUNK
    c_h = pltpu.async_copy(matches_hbm.at[0, pl.ds(base, _CHUNK)], idx_h, s_h)
    c_a = pltpu.async_copy(matches_hbm.at[1, pl.ds(base, _CHUNK)], idx_a, s_a)
    c_h.wait()
    g_h0 = pltpu.async_copy(
        rating_hbm.at[idx_h.at[pl.ds(0, _HALF)]],
        gath_h.at[pl.ds(0, _HALF)], s_h)
    c_a.wait()
    g_a0 = pltpu.async_copy(
        rating_hbm.at[idx_a.at[pl.ds(0, _HALF)]],
        gath_a.at[pl.ds(0, _HALF)], s_a)
    g_h1 = pltpu.async_copy(
        rating_hbm.at[idx_h.at[pl.ds(_HALF, _HALF)]],
        gath_h.at[pl.ds(_HALF, _HALF)], s_h)
    g_a1 = pltpu.async_copy(
        rating_hbm.at[idx_a.at[pl.ds(_HALF, _HALF)]],
        gath_a.at[pl.ds(_HALF, _HALF)], s_a)
    g_h0.wait()
    g_a0.wait()
    g_h1.wait()
    g_a1.wait()

    def body(i, _):
        off = i * _LANES
        hr = gath_h[pl.ds(off, _LANES)]
        ar = gath_a[pl.ds(off, _LANES)]
        out_v[pl.ds(off, _LANES)] = 1.0 / (1.0 + jnp.exp((ar - hr) * _K))
        return 0

    lax.fori_loop(0, _CHUNK // _LANES, body, 0)
    pltpu.async_copy(out_v, out_hbm.at[pl.ds(base, _CHUNK)], s_h).wait()


def kernel(matches, rating):
    return _elo_sc(matches, rating)


# speedup vs baseline: 1.0070x; 1.0070x over previous
import functools
import math

import jax
import jax.numpy as jnp
from jax import lax
from jax.experimental import pallas as pl
from jax.experimental.pallas import tpu as pltpu
from jax.experimental.pallas import tpu_sc as plsc

_BATCH = 16384
_C = 10.0
_D = 400.0
_K = math.log(_C) / _D

_NUM_WORKERS = 32
_CHUNK = _BATCH // _NUM_WORKERS
_LANES = 16


@functools.partial(
    pl.kernel,
    mesh=plsc.VectorSubcoreMesh(core_axis_name="c", subcore_axis_name="s"),
    out_type=jax.ShapeDtypeStruct((_BATCH,), jnp.float32),
    scratch_types=[
        pltpu.VMEM((_CHUNK,), jnp.int32),
        pltpu.VMEM((_CHUNK,), jnp.int32),
        pltpu.VMEM((_CHUNK,), jnp.float32),
        pltpu.VMEM((_CHUNK,), jnp.float32),
        pltpu.VMEM((_CHUNK,), jnp.float32),
        pltpu.SemaphoreType.DMA,
        pltpu.SemaphoreType.DMA,
    ],
)
def _elo_sc(matches_hbm, rating_hbm, out_hbm,
            idx_h, idx_a, gath_h, gath_a, out_v, s_h, s_a):
    wid = lax.axis_index("s") * 2 + lax.axis_index("c")
    base = wid * _CHUNK
    c_h = pltpu.async_copy(matches_hbm.at[0, pl.ds(base, _CHUNK)], idx_h, s_h)
    c_a = pltpu.async_copy(matches_hbm.at[1, pl.ds(base, _CHUNK)], idx_a, s_a)
    c_h.wait()
    g_h = pltpu.async_copy(rating_hbm.at[idx_h], gath_h, s_h)
    c_a.wait()
    g_a = pltpu.async_copy(rating_hbm.at[idx_a], gath_a, s_a)
    g_h.wait()
    g_a.wait()

    def body(i, _):
        off = i * _LANES
        hr = gath_h[pl.ds(off, _LANES)]
        ar = gath_a[pl.ds(off, _LANES)]
        out_v[pl.ds(off, _LANES)] = 1.0 / (1.0 + jnp.exp((ar - hr) * _K))
        return 0

    lax.fori_loop(0, _CHUNK // _LANES, body, 0)
    pltpu.async_copy(out_v, out_hbm.at[pl.ds(base, _CHUNK)], s_h).wait()


def kernel(matches, rating):
    return _elo_sc(matches, rating)
